# Initial kernel scaffold; baseline (speedup 1.0000x reference)
import jax
import jax.numpy as jnp
from jax.experimental import pallas as pl


def kernel(embeddings, labels):
    raise NotImplementedError("write your pallas kernel here")


# speedup vs baseline: 2.5901x; 2.5901x over previous
import functools

import jax
import jax.numpy as jnp
from jax import lax
from jax.experimental import pallas as pl
from jax.experimental.pallas import tpu as pltpu

_TEMPERATURE = 0.07
_EPS = 1e-8
_BM = 512
_LANES = 128
_SCALE = 1.4426950408889634 / _TEMPERATURE


def _loss_kernel(emb_hbm, labr_ref, labc_ref, loss_ref, cnt_ref,
                 raw_ref, en_ref, ens_ref, sem, *, nj, bm, d):
    s = pl.program_id(1)
    i = pl.program_id(0) * pl.num_programs(1) + s

    @pl.when(s == 0)
    def _prologue():
        cp = pltpu.make_async_copy(emb_hbm, raw_ref, sem)
        cp.start()
        cp.wait()
        for b in range(nj):
            x = raw_ref[b]
            e = x * lax.rsqrt(jnp.sum(x * x, axis=1, keepdims=True))
            en_ref[b] = e.astype(jnp.bfloat16)
            ens_ref[b] = (e * jnp.float32(_SCALE)).astype(jnp.bfloat16)

    lr = labr_ref[...]
    num_acc = jnp.zeros((bm, _LANES), jnp.float32)
    den_acc = jnp.zeros((bm, _LANES), jnp.float32)
    erow = ens_ref[i]
    for t in range(nj):
        j = i if t == 0 else lax.rem(i + t, nj)
        eblk = en_ref[j]
        sim = lax.dot_general(
            erow, eblk, (((1,), (1,)), ((), ())),
            preferred_element_type=jnp.float32)
        lc = labc_ref[j]
        for c in range(bm // _LANES):
            sl = slice(c * _LANES, (c + 1) * _LANES)
            sim_c = sim[:, sl]
            ex_c = jnp.exp2(sim_c)
            pos_c = jnp.where(sim_c > 0, ex_c, 0.0)
            num_c = jnp.where(lr == lc[:, sl], pos_c, 0.0)
            if t == 0:
                rows = lax.broadcasted_iota(jnp.int32, (bm, _LANES), 0)
                cols = lax.broadcasted_iota(jnp.int32, (bm, _LANES), 1) + c * _LANES
                ndiag = rows != cols
                num_c = jnp.where(ndiag, num_c, 0.0)
                den_acc = den_acc + jnp.where(ndiag, ex_c, 0.0)
            else:
                den_acc = den_acc + ex_c
            num_acc = num_acc + num_c
    num_row = jnp.sum(num_acc, axis=1, keepdims=True)
    den_row = jnp.sum(den_acc, axis=1, keepdims=True)
    rvalid = (num_row > 0.0) & (den_row > 0.0)
    num_s = jnp.where(rvalid, num_row, 1.0)
    den_s = jnp.where(rvalid, den_row, 1.0)
    li = -jnp.log(num_s / (den_s + _EPS))
    li = jnp.where(rvalid, li, 0.0)
    loss_ref[...] = jnp.sum(li, axis=0, keepdims=True)[None]
    cnt_ref[...] = jnp.sum(rvalid.astype(jnp.float32), axis=0, keepdims=True)[None]


def kernel(embeddings, labels):
    n, d = embeddings.shape
    bm = _BM if n % (2 * _BM) == 0 else n
    nj = n // bm
    half = max(nj // 2, 1)
    cores = nj // half

    labf = labels.astype(jnp.float32)
    labr = jnp.broadcast_to(labf[:, None], (n, _LANES))
    labc = labf.reshape(nj, 1, bm)

    loss_sums, cnts = pl.pallas_call(
        functools.partial(_loss_kernel, nj=nj, bm=bm, d=d),
        grid=(cores, half),
        in_specs=[
            pl.BlockSpec(memory_space=pl.ANY),
            pl.BlockSpec((bm, _LANES), lambda c, s: (c * half + s, 0)),
            pl.BlockSpec((nj, 1, bm), lambda c, s: (0, 0, 0)),
        ],
        out_specs=[
            pl.BlockSpec((1, 1, 1), lambda c, s: (c * half + s, 0, 0)),
            pl.BlockSpec((1, 1, 1), lambda c, s: (c * half + s, 0, 0)),
        ],
        out_shape=[
            jax.ShapeDtypeStruct((nj, 1, 1), jnp.float32),
            jax.ShapeDtypeStruct((nj, 1, 1), jnp.float32),
        ],
        scratch_shapes=[
            pltpu.VMEM((nj, bm, d), jnp.float32),
            pltpu.VMEM((nj, bm, d), jnp.bfloat16),
            pltpu.VMEM((nj, bm, d), jnp.bfloat16),
            pltpu.SemaphoreType.DMA,
        ],
        compiler_params=pltpu.CompilerParams(
            dimension_semantics=("parallel", "arbitrary"),
            vmem_limit_bytes=100 * 1024 * 1024,
        ),
    )(embeddings.astype(jnp.float32).reshape(nj, bm, d), labr, labc)

    total = jnp.sum(loss_sums)
    cnt = jnp.sum(cnts)
    mean = total / jnp.maximum(cnt, 1.0)
    return jnp.abs(jnp.where(cnt > 0.0, mean, 0.0))
